# Initial kernel scaffold; baseline (speedup 1.0000x reference)
import jax
import jax.numpy as jnp
from jax.experimental import pallas as pl


def kernel(anchors, deltas, scores, img_info):
    raise NotImplementedError("write your pallas kernel here")


# speedup vs baseline: 34.7050x; 34.7050x over previous
import functools

import jax
import jax.numpy as jnp
from jax import lax
from jax.experimental import pallas as pl
from jax.experimental.pallas import tpu as pltpu

B = 4
N = 20000
ROWS = 160
LANES = 128
NPAD = ROWS * LANES
PRE = 6000
POST = 1000
TH = 0.7
C = 512
NTILES = (PRE + C - 1) // C
KCH = 12

_F32 = jnp.float32
_DOT = dict(precision=lax.Precision.HIGHEST, preferred_element_type=_F32)


def _tdot(a, b, ca, cb):
    return lax.dot_general(a, b, (((ca,), (cb,)), ((), ())), **_DOT)


def _excl_prefix(x, axis, size):
    inc = x
    sh = 1
    while sh < size:
        zshape = list(inc.shape)
        zshape[axis] = sh
        idx = [slice(None)] * inc.ndim
        idx[axis] = slice(0, inc.shape[axis] - sh)
        inc = inc + jnp.concatenate(
            [jnp.zeros(zshape, inc.dtype), inc[tuple(idx)]], axis=axis)
        sh *= 2
    return inc - x


def _nms_body(ax1r, ay1r, ax2r, ay2r, dxr, dyr, dwr, dhr, scr, hm1r, wm1r,
              ox1r, oy1r, ox2r, oy2r, kcntr):
    ax1 = ax1r[...]
    ay1 = ay1r[...]
    ax2 = ax2r[...]
    ay2 = ay2r[...]
    aw = ax2 - ax1 + 1.0
    ah = ay2 - ay1 + 1.0
    acx = ax1 + 0.5 * aw
    acy = ay1 + 0.5 * ah
    dx = dxr[...]
    dy = dyr[...]
    dw = dwr[...]
    dh = dhr[...]
    pcx = dx * aw + acx
    pcy = dy * ah + acy
    pw = jnp.exp(dw) * aw
    ph = jnp.exp(dh) * ah
    hm1 = hm1r[...][:, :, 0:1]
    wm1 = wm1r[...][:, :, 0:1]
    zero = jnp.zeros((), _F32)
    x1 = jnp.clip(pcx - 0.5 * pw, zero, wm1)
    y1 = jnp.clip(pcy - 0.5 * ph, zero, hm1)
    x2 = jnp.clip(pcx + 0.5 * pw - 1.0, zero, wm1)
    y2 = jnp.clip(pcy + 0.5 * ph - 1.0, zero, hm1)
    area = (x2 - x1 + 1.0) * (y2 - y1 + 1.0)

    s = scr[...]
    bits = lax.bitcast_convert_type(s, jnp.int32)
    lin = (lax.broadcasted_iota(jnp.int32, (B, ROWS, LANES), 1) * LANES
           + lax.broadcasted_iota(jnp.int32, (B, ROWS, LANES), 2))

    def cnt3(mask):
        c = jnp.sum(mask.astype(_F32), axis=2, keepdims=True)
        return jnp.sum(c, axis=1, keepdims=True)

    ranks = [float(min((t + 1) * C, PRE)) for t in range(NTILES)]
    taus = [jnp.zeros((B, 1, 1), jnp.int32) for _ in range(NTILES)]
    for bit in range(30, -1, -1):
        for t in range(NTILES):
            tt = taus[t] | (1 << bit)
            taus[t] = jnp.where(cnt3(bits >= tt) >= ranks[t], tt, taus[t])
    cuts = []
    for t in range(NTILES):
        eq = bits == taus[t]
        m = ranks[t] - cnt3(bits > taus[t])
        lo = jnp.zeros((B, 1, 1), jnp.int32)
        hi = jnp.full((B, 1, 1), NPAD, jnp.int32)
        for _ in range(15):
            mid = (lo + hi) // 2
            ok = cnt3(eq & (lin <= mid)) >= m
            hi = jnp.where(ok, mid, hi)
            lo = jnp.where(ok, lo, mid + 1)
        cuts.append(lo)

    def cum_mask(t):
        return (bits > taus[t]) | ((bits == taus[t]) & (lin <= cuts[t]))

    zout = jnp.zeros((B, KCH, LANES), _F32)
    ox1r[...] = zout
    oy1r[...] = zout
    ox2r[...] = zout
    oy2r[...] = zout
    kcntr[...] = jnp.zeros((B, 1, LANES), _F32)

    lane1 = lax.broadcasted_iota(jnp.int32, (1, LANES), 1).astype(_F32)
    crow = lax.broadcasted_iota(jnp.int32, (C, 1), 0).astype(_F32)
    clane = lax.broadcasted_iota(jnp.int32, (1, C), 1).astype(_F32)
    eyeR = (lax.broadcasted_iota(jnp.int32, (ROWS, ROWS), 0)
            == lax.broadcasted_iota(jnp.int32, (ROWS, ROWS), 1)).astype(_F32)
    eyeC = (lax.broadcasted_iota(jnp.int32, (C, C), 0)
            == lax.broadcasted_iota(jnp.int32, (C, C), 1)).astype(_F32)

    prev_mask = [None]

    for t in range(NTILES):
        go = jnp.min(kcntr[...]) < float(POST)

        @pl.when(go)
        def _tile():
            mcur = cum_mask(t)
            if t == 0:
                tile = mcur
            else:
                tile = mcur & jnp.logical_not(prev_mask[0])
            intile = tile.astype(_F32)
            rowcnt = jnp.sum(intile, axis=2, keepdims=True)
            base = _excl_prefix(rowcnt, 1, ROWS)
            lanerank = _excl_prefix(intile, 2, LANES)

            for b in range(B):
                kcv = kcntr[b, 0:1, :]
                kcnt_b = jnp.max(kcv)
                base_l = _tdot(base[b], eyeR, 0, 0)
                cnt_l = _tdot(rowcnt[b], eyeR, 0, 0)
                oh = ((base_l <= crow)
                      & (crow < base_l + cnt_l)).astype(_F32)
                gx1 = _tdot(oh, x1[b], 1, 0)
                gy1 = _tdot(oh, y1[b], 1, 0)
                gx2 = _tdot(oh, x2[b], 1, 0)
                gy2 = _tdot(oh, y2[b], 1, 0)
                gsc = _tdot(oh, s[b], 1, 0)
                glr = _tdot(oh, lanerank[b], 1, 0)
                git = _tdot(oh, intile[b], 1, 0)
                rbase = jnp.sum(oh * base_l, axis=1, keepdims=True)
                want = crow - rbase
                pickl = ((glr == want) & (git > 0.5)).astype(_F32)
                tx1 = jnp.sum(gx1 * pickl, axis=1, keepdims=True)
                ty1 = jnp.sum(gy1 * pickl, axis=1, keepdims=True)
                tx2 = jnp.sum(gx2 * pickl, axis=1, keepdims=True)
                ty2 = jnp.sum(gy2 * pickl, axis=1, keepdims=True)
                tsc = jnp.sum(gsc * pickl, axis=1, keepdims=True)
                tvalid = jnp.sum(pickl, axis=1, keepdims=True) > 0.5
                tar = (tx2 - tx1 + 1.0) * (ty2 - ty1 + 1.0)

                killed = jnp.zeros((C, 1), _F32)
                for kc in range(KCH):
                    kx1 = ox1r[b, kc:kc + 1, :]
                    ky1 = oy1r[b, kc:kc + 1, :]
                    kx2 = ox2r[b, kc:kc + 1, :]
                    ky2 = oy2r[b, kc:kc + 1, :]
                    kval = (lane1 + float(kc * LANES)) < kcnt_b
                    xx1 = jnp.maximum(tx1, kx1)
                    yy1 = jnp.maximum(ty1, ky1)
                    xx2 = jnp.minimum(tx2, kx2)
                    yy2 = jnp.minimum(ty2, ky2)
                    inter = (jnp.maximum(xx2 - xx1 + 1.0, zero)
                             * jnp.maximum(yy2 - yy1 + 1.0, zero))
                    ka = (kx2 - kx1 + 1.0) * (ky2 - ky1 + 1.0)
                    iou = inter / (ka + tar - inter)
                    hit = ((iou > TH) & kval).astype(_F32)
                    killed = killed + jnp.sum(hit, axis=1, keepdims=True)
                alive0 = (tvalid & (killed < 0.5)).astype(_F32)

                lx1 = _tdot(tx1, eyeC, 0, 0)
                ly1 = _tdot(ty1, eyeC, 0, 0)
                lx2 = _tdot(tx2, eyeC, 0, 0)
                ly2 = _tdot(ty2, eyeC, 0, 0)
                lsc = _tdot(tsc, eyeC, 0, 0)
                lar = (lx2 - lx1 + 1.0) * (ly2 - ly1 + 1.0)
                xx1 = jnp.maximum(tx1, lx1)
                yy1 = jnp.maximum(ty1, ly1)
                xx2 = jnp.minimum(tx2, lx2)
                yy2 = jnp.minimum(ty2, ly2)
                inter = (jnp.maximum(xx2 - xx1 + 1.0, zero)
                         * jnp.maximum(yy2 - yy1 + 1.0, zero))
                iou = inter / (tar + lar - inter)
                prec = (tsc > lsc) | ((tsc == lsc) & (crow < clane))
                smat = ((iou > TH) & prec).astype(_F32)

                al0 = _tdot(alive0, eyeC, 0, 0)

                def fcond(carry):
                    return carry[1]

                def fbody(carry):
                    kv, _ = carry
                    supc = _tdot(kv, smat, 1, 0)
                    kn = al0 * (supc < 0.5).astype(_F32)
                    chg = jnp.sum(jnp.abs(kn - kv)) > 0.0
                    return kn, chg

                keep_l, _ = lax.while_loop(
                    fcond, fbody, (al0, jnp.sum(al0) >= 0.0))

                keep_r = _tdot(eyeC, keep_l, 1, 1)
                rank_l = _tdot(keep_l, prec.astype(_F32), 1, 0)
                slot = kcnt_b + _tdot(eyeC, rank_l, 1, 1)
                for kc in range(KCH):
                    lane_g = lane1 + float(kc * LANES)
                    ohk = ((slot == lane_g) & (keep_r > 0.5)).astype(_F32)
                    ox1r[b, kc:kc + 1, :] += jnp.sum(ohk * tx1, axis=0,
                                                     keepdims=True)
                    oy1r[b, kc:kc + 1, :] += jnp.sum(ohk * ty1, axis=0,
                                                     keepdims=True)
                    ox2r[b, kc:kc + 1, :] += jnp.sum(ohk * tx2, axis=0,
                                                     keepdims=True)
                    oy2r[b, kc:kc + 1, :] += jnp.sum(ohk * ty2, axis=0,
                                                     keepdims=True)
                kcntr[b, 0:1, :] = jnp.broadcast_to(
                    kcnt_b + jnp.sum(keep_r), (1, LANES))

        prev_mask[0] = cum_mask(t)


@jax.jit
def kernel(anchors, deltas, scores, img_info):
    f32 = _F32
    apad = jnp.pad(anchors.astype(f32), ((0, NPAD - N), (0, 0)))
    ax1 = apad[:, 0].reshape(ROWS, LANES)
    ay1 = apad[:, 1].reshape(ROWS, LANES)
    ax2 = apad[:, 2].reshape(ROWS, LANES)
    ay2 = apad[:, 3].reshape(ROWS, LANES)
    dpad = jnp.pad(deltas.astype(f32), ((0, 0), (0, NPAD - N), (0, 0)))
    dx = dpad[:, :, 0].reshape(B, ROWS, LANES)
    dy = dpad[:, :, 1].reshape(B, ROWS, LANES)
    dw = dpad[:, :, 2].reshape(B, ROWS, LANES)
    dh = dpad[:, :, 3].reshape(B, ROWS, LANES)
    sc = jnp.pad(scores.astype(f32), ((0, 0), (0, NPAD - N)),
                 constant_values=-1.0).reshape(B, ROWS, LANES)
    hm1 = jnp.broadcast_to((img_info[:, 0] - 1.0)[:, None, None],
                           (B, 1, LANES)).astype(f32)
    wm1 = jnp.broadcast_to((img_info[:, 1] - 1.0)[:, None, None],
                           (B, 1, LANES)).astype(f32)

    out_shapes = [jax.ShapeDtypeStruct((B, KCH, LANES), f32)] * 4
    res = pl.pallas_call(
        _nms_body,
        out_shape=tuple(out_shapes),
        scratch_shapes=[pltpu.VMEM((B, 1, LANES), f32)],
    )(ax1, ay1, ax2, ay2, dx, dy, dw, dh, sc, hm1, wm1)
    x1o, y1o, x2o, y2o = [r.reshape(B, KCH * LANES)[:, :POST] for r in res]
    boxes = jnp.stack([x1o, y1o, x2o, y2o], axis=-1)
    col0 = jnp.broadcast_to(
        jnp.arange(B, dtype=f32)[:, None, None], (B, POST, 1))
    return jnp.concatenate([col0, boxes], axis=-1)
